# Initial kernel scaffold; baseline (speedup 1.0000x reference)
import jax
import jax.numpy as jnp
from jax.experimental import pallas as pl


def kernel(x, edge_index, W0, b0, pw1, Wd1, bd1, pw2, Wd2, bd2, pw3, Wd3, bd3, Wu1, bu1, Wu2, bu2, Wu3, bu3):
    raise NotImplementedError("write your pallas kernel here")


# speedup vs baseline: 1.1921x; 1.1921x over previous
import math
import functools

import jax
import jax.numpy as jnp
from jax import lax
from jax.experimental import pallas as pl


def _gcn_epilogue_body(s_ref, y_ref, dinv_ref, coef_ref, b_ref, o_ref, *, relu):
    dinv = dinv_ref[...][:, None]
    coef = coef_ref[...][:, None]
    out = dinv * s_ref[...] + coef * y_ref[...] + b_ref[...][None, :]
    if relu:
        out = jnp.maximum(out, 0.0)
    o_ref[...] = out


def _gcn_epilogue(S, y, dinv, coef, b, relu):
    n, dd = S.shape
    return pl.pallas_call(
        functools.partial(_gcn_epilogue_body, relu=relu),
        out_shape=jax.ShapeDtypeStruct((n, dd), jnp.float32),
    )(S, y, dinv, coef, b)


def _dense_gcn(A, y, dinv, bias, relu):
    z = dinv[:, None] * y
    S = A.T @ z
    coef = 2.0 * dinv * dinv
    return _gcn_epilogue(S, y, dinv, coef, bias, relu)


def _topk(score, k):
    vals, perm = lax.top_k(score, k)
    n = score.shape[0]
    invp = jnp.full((n,), -1, jnp.int32).at[perm].set(
        jnp.arange(k, dtype=jnp.int32))
    return vals, perm, invp


def _zero_diag(P):
    n = P.shape[0]
    idx = jnp.arange(n)
    return P.at[idx, idx].set(0.0)


def kernel(x, edge_index, W0, b0, pw1, Wd1, bd1, pw2, Wd2, bd2, pw3, Wd3,
           bd3, Wu1, bu1, Wu2, bu2, Wu3, bu3):
    n0 = x.shape[1]
    e = edge_index.shape[1]
    f32 = jnp.float32

    gn = jnp.sqrt(jnp.mean(x * x))

    s = edge_index[0]
    d = edge_index[1]
    self_e = (s == d)

    ones_e = jnp.ones((e,), f32)
    colcount = jnp.zeros((n0,), f32).at[d].add(ones_e)
    selfcnt = jnp.zeros((n0,), f32).at[d].add(jnp.where(self_e, 1.0, 0.0))
    no_self = (selfcnt == 0.0).astype(f32)
    deg0 = colcount + 2.0 * no_self
    dinv0 = lax.rsqrt(deg0)
    coef0 = 2.0 * no_self * dinv0 * dinv0

    outs = []
    for bidx in range(x.shape[0]):
        xn = x[bidx] / gn

        y0 = xn @ W0
        z0 = dinv0[:, None] * y0
        S0 = jnp.zeros((n0, y0.shape[1]), f32).at[d].add(z0[s])
        x1 = _gcn_epilogue(S0, y0, dinv0, coef0, b0, True)

        k1 = (n0 + 1) // 2
        score1 = jnp.tanh((x1 @ pw1) / jnp.linalg.norm(pw1))
        vals1, perm1, invp1 = _topk(score1, k1)
        xp1 = x1[perm1] * vals1[:, None]

        is1 = invp1[s]
        id1 = invp1[d]
        wrow = jnp.where((~self_e) & (is1 >= 0), 1.0, 0.0)
        wcol = jnp.where((~self_e) & (id1 >= 0), 1.0, 0.0)
        Brows = jnp.zeros((k1, n0), f32).at[is1, d].add(wrow)
        Brows = Brows.at[jnp.arange(k1), perm1].add(1.0)
        Bcols = jnp.zeros((n0, k1), f32).at[s, id1].add(wcol)
        Bcols = Bcols.at[perm1, jnp.arange(k1)].add(1.0)
        A1p = _zero_diag(Brows @ Bcols)
        deg1 = A1p.sum(axis=0) + 2.0
        dinv1 = lax.rsqrt(deg1)

        x2 = _dense_gcn(A1p, xp1 @ Wd1, dinv1, bd1, True)

        k2 = (k1 + 1) // 2
        score2 = jnp.tanh((x2 @ pw2) / jnp.linalg.norm(pw2))
        vals2, perm2, invp2 = _topk(score2, k2)
        xp2 = x2[perm2] * vals2[:, None]
        G1 = A1p.at[jnp.arange(k1), jnp.arange(k1)].set(1.0)
        A2p = _zero_diag(G1[perm2, :] @ G1[:, perm2])
        deg2 = A2p.sum(axis=0) + 2.0
        dinv2 = lax.rsqrt(deg2)

        x3 = _dense_gcn(A2p, xp2 @ Wd2, dinv2, bd2, True)

        k3 = (k2 + 1) // 2
        score3 = jnp.tanh((x3 @ pw3) / jnp.linalg.norm(pw3))
        vals3, perm3, invp3 = _topk(score3, k3)
        xp3 = x3[perm3] * vals3[:, None]
        G2 = A2p.at[jnp.arange(k2), jnp.arange(k2)].set(1.0)
        A3p = _zero_diag(G2[perm3, :] @ G2[:, perm3])
        deg3 = A3p.sum(axis=0) + 2.0
        dinv3 = lax.rsqrt(deg3)

        x4 = _dense_gcn(A3p, xp3 @ Wd3, dinv3, bd3, True)

        up = jnp.where((invp3 >= 0)[:, None],
                       x4[jnp.maximum(invp3, 0)], 0.0)
        xu = x3 + up
        xu = _dense_gcn(A2p, xu @ Wu1, dinv2, bu1, True)

        up = jnp.where((invp2 >= 0)[:, None],
                       xu[jnp.maximum(invp2, 0)], 0.0)
        xu = x2 + up
        xu = _dense_gcn(A1p, xu @ Wu2, dinv1, bu2, True)

        up = jnp.where((invp1 >= 0)[:, None],
                       xu[jnp.maximum(invp1, 0)], 0.0)
        xu = x1 + up
        yf = xu @ Wu3
        zf = dinv0[:, None] * yf
        Sf = jnp.zeros((n0, yf.shape[1]), f32).at[d].add(zf[s])
        out = _gcn_epilogue(Sf, yf, dinv0, coef0, bu3, False)
        outs.append(out)

    return jnp.stack(outs, 0)


# speedup vs baseline: 1.4326x; 1.2018x over previous
import functools

import jax
import jax.numpy as jnp
from jax import lax
from jax.experimental import pallas as pl
from jax.experimental.pallas import tpu as pltpu

f32 = jnp.float32
bf16 = jnp.bfloat16


def _padto(n):
    m = 1280 if n >= 1280 else 128
    return ((n + m - 1) // m) * m


def _pick(n, pref):
    return pref if n % pref == 0 else n


def _mm_body(a_ref, b_ref, *refs, nk, dims, acc_dtype, out_dtype, zero_diag,
             want_csum, want_ct, ct_dtype, bm, bn):
    if want_csum and want_ct:
        c_ref, csum_ref, ct_ref, acc_ref = refs
    elif want_csum:
        c_ref, csum_ref, acc_ref = refs
        ct_ref = None
    elif want_ct:
        c_ref, ct_ref, acc_ref = refs
        csum_ref = None
    else:
        c_ref, acc_ref = refs
        csum_ref = ct_ref = None

    j = pl.program_id(0)
    i = pl.program_id(1)
    k = pl.program_id(2)

    a = a_ref[...].astype(acc_dtype)
    b = b_ref[...].astype(acc_dtype)
    part = lax.dot_general(a, b, (((dims[0],), (dims[1],)), ((), ())),
                           preferred_element_type=f32)

    @pl.when(k == 0)
    def _():
        acc_ref[...] = part

    @pl.when(k > 0)
    def _():
        acc_ref[...] += part

    @pl.when(k == nk - 1)
    def _():
        acc = acc_ref[...]
        if zero_diag:
            rr = lax.broadcasted_iota(jnp.int32, (bm, bn), 0) + i * bm
            cc = lax.broadcasted_iota(jnp.int32, (bm, bn), 1) + j * bn
            acc = jnp.where(rr == cc, 0.0, acc)
        c_ref[...] = acc.astype(out_dtype)
        if csum_ref is not None:
            s = jnp.sum(acc, axis=0, keepdims=True)

            @pl.when(i == 0)
            def _():
                csum_ref[...] = s

            @pl.when(i > 0)
            def _():
                csum_ref[...] += s

        if ct_ref is not None:
            ct_ref[...] = acc.T.astype(ct_dtype)


def _matmul(a, b, *, dims=(1, 0), bm=None, bn=None, bk=None, acc_dtype=bf16,
            out_dtype=f32, zero_diag=False, want_csum=False, want_ct=False,
            ct_dtype=bf16):
    ma = 1 - dims[0]
    mb = 1 - dims[1]
    M = a.shape[ma]
    K = a.shape[dims[0]]
    N = b.shape[mb]
    bm = _pick(M, 1280) if bm is None else bm
    bn = _pick(N, 1280) if bn is None else bn
    bk = _pick(K, 2560) if bk is None else bk
    assert M % bm == 0 and N % bn == 0 and K % bk == 0, (
        a.shape, b.shape, bm, bn, bk)
    ni, nj, nk = M // bm, N // bn, K // bk

    if dims[0] == 1:
        a_spec = pl.BlockSpec((bm, bk), lambda j, i, k: (i, k))
    else:
        a_spec = pl.BlockSpec((bk, bm), lambda j, i, k: (k, i))
    if dims[1] == 0:
        b_spec = pl.BlockSpec((bk, bn), lambda j, i, k: (k, j))
    else:
        b_spec = pl.BlockSpec((bn, bk), lambda j, i, k: (j, k))

    out_shape = [jax.ShapeDtypeStruct((M, N), out_dtype)]
    out_specs = [pl.BlockSpec((bm, bn), lambda j, i, k: (i, j))]
    if want_csum:
        out_shape.append(jax.ShapeDtypeStruct((1, N), f32))
        out_specs.append(pl.BlockSpec((1, bn), lambda j, i, k: (0, j)))
    if want_ct:
        out_shape.append(jax.ShapeDtypeStruct((N, M), ct_dtype))
        out_specs.append(pl.BlockSpec((bn, bm), lambda j, i, k: (j, i)))

    body = functools.partial(
        _mm_body, nk=nk, dims=dims, acc_dtype=acc_dtype, out_dtype=out_dtype,
        zero_diag=zero_diag, want_csum=want_csum, want_ct=want_ct,
        ct_dtype=ct_dtype, bm=bm, bn=bn)
    outs = pl.pallas_call(
        body,
        grid=(nj, ni, nk),
        in_specs=[a_spec, b_spec],
        out_specs=out_specs,
        out_shape=out_shape,
        scratch_shapes=[pltpu.VMEM((bm, bn), f32)],
    )(a, b)
    return outs if (want_csum or want_ct) else outs[0]


def _cast_body(x_ref, o_ref, dt):
    o_ref[...] = x_ref[...].astype(dt)


def _cast_bf16(x):
    n, m = x.shape
    rb = n
    for cand in (1024, 512, 256, 128):
        if n % cand == 0 and cand * m * 12 < 45_000_000:
            rb = cand
            break
    return pl.pallas_call(
        functools.partial(_cast_body, dt=bf16),
        grid=(n // rb,),
        in_specs=[pl.BlockSpec((rb, m), lambda i: (i, 0))],
        out_specs=pl.BlockSpec((rb, m), lambda i: (i, 0)),
        out_shape=jax.ShapeDtypeStruct((n, m), bf16),
    )(x)


def _gcn_epilogue_body(s_ref, y_ref, dinv_ref, coef_ref, b_ref, o_ref, *,
                       relu):
    dinv = dinv_ref[...][:, None]
    coef = coef_ref[...][:, None]
    out = dinv * s_ref[...] + coef * y_ref[...] + b_ref[...][None, :]
    if relu:
        out = jnp.maximum(out, 0.0)
    o_ref[...] = out


def _gcn_epilogue(S, y, dinv, coef, b, relu):
    n, dd = S.shape
    return pl.pallas_call(
        functools.partial(_gcn_epilogue_body, relu=relu),
        out_shape=jax.ShapeDtypeStruct((n, dd), f32),
    )(S, y, dinv, coef, b)


def _feature_mm(x, W):
    return _matmul(x, W, bm=x.shape[0], bn=W.shape[1], bk=x.shape[1],
                   acc_dtype=f32, out_dtype=f32)


def _dense_gcn(A, y, dinv, bias, relu):
    npad = A.shape[0]
    n = y.shape[0]
    z = dinv[:, None] * y
    zp = jnp.zeros((npad, y.shape[1]), f32).at[:n].set(z)
    S = _matmul(A, zp, dims=(0, 0), bm=_pick(npad, 1280), bn=y.shape[1],
                bk=_pick(npad, 1280), acc_dtype=f32, out_dtype=f32)[:n]
    coef = 2.0 * dinv * dinv
    return _gcn_epilogue(S, y, dinv, coef, bias, relu)


def _topk(score, k):
    vals, perm = lax.top_k(score, k)
    n = score.shape[0]
    invp = jnp.full((n,), -1, jnp.int32).at[perm].set(
        jnp.arange(k, dtype=jnp.int32))
    return vals, perm, invp


def _pool_adj(A, AT, perm, kpad):
    k = perm.shape[0]
    permp = jnp.full((kpad,), -1, jnp.int32).at[:k].set(perm)
    onehot = (permp[:, None] == jnp.arange(A.shape[1])[None, :]).astype(
        A.dtype)
    U = A[permp, :] + onehot
    W = AT[permp, :] + onehot
    return U, W


def kernel(x, edge_index, W0, b0, pw1, Wd1, bd1, pw2, Wd2, bd2, pw3, Wd3,
           bd3, Wu1, bu1, Wu2, bu2, Wu3, bu3):
    n0 = x.shape[1]
    e = edge_index.shape[1]
    n0p = _padto(n0)
    k1 = (n0 + 1) // 2
    k2 = (k1 + 1) // 2
    k3 = (k2 + 1) // 2
    k1p, k2p, k3p = _padto(k1), _padto(k2), _padto(k3)

    gn = jnp.sqrt(jnp.mean(x * x))

    s = edge_index[0]
    d = edge_index[1]
    self_e = (s == d)

    ones_e = jnp.ones((e,), f32)
    colcount = jnp.zeros((n0,), f32).at[d].add(ones_e)
    selfcnt = jnp.zeros((n0,), f32).at[d].add(jnp.where(self_e, 1.0, 0.0))
    no_self = (selfcnt == 0.0).astype(f32)
    deg0 = colcount + 2.0 * no_self
    dinv0 = lax.rsqrt(deg0)
    coef0 = 2.0 * no_self * dinv0 * dinv0

    outs = []
    for bidx in range(x.shape[0]):
        xn = x[bidx] / gn

        y0 = _feature_mm(xn, W0)
        z0 = dinv0[:, None] * y0
        S0 = jnp.zeros((n0, y0.shape[1]), f32).at[d].add(z0[s])
        x1 = _gcn_epilogue(S0, y0, dinv0, coef0, b0, True)

        score1 = jnp.tanh((x1 @ pw1) / jnp.linalg.norm(pw1))
        vals1, perm1, invp1 = _topk(score1, k1)
        xp1 = x1[perm1] * vals1[:, None]

        is1 = invp1[s]
        id1 = invp1[d]
        wrow = jnp.where((~self_e) & (is1 >= 0), 1.0, 0.0)
        wcol = jnp.where((~self_e) & (id1 >= 0), 1.0, 0.0)
        Brows = jnp.zeros((k1p, n0p), f32).at[is1, d].add(wrow)
        Brows = Brows.at[jnp.arange(k1), perm1].add(1.0)
        Bcols = jnp.zeros((n0p, k1p), f32).at[s, id1].add(wcol)
        Bcols = Bcols.at[perm1, jnp.arange(k1)].add(1.0)
        A1p, csum1, A1pT = _matmul(
            _cast_bf16(Brows), _cast_bf16(Bcols), acc_dtype=bf16,
            out_dtype=bf16, zero_diag=True, want_csum=True, want_ct=True)
        deg1 = csum1[0, :k1] + 2.0
        dinv1 = lax.rsqrt(deg1)

        x2 = _dense_gcn(A1p, _feature_mm(xp1, Wd1), dinv1, bd1, True)

        score2 = jnp.tanh((x2 @ pw2) / jnp.linalg.norm(pw2))
        vals2, perm2, invp2 = _topk(score2, k2)
        xp2 = x2[perm2] * vals2[:, None]
        U2, W2 = _pool_adj(A1p, A1pT, perm2, k2p)
        A2p, csum2, A2pT = _matmul(
            U2, W2, dims=(1, 1), bm=_pick(k2p, 640), bn=_pick(k2p, 640),
            acc_dtype=bf16, out_dtype=f32, zero_diag=True, want_csum=True,
            want_ct=True, ct_dtype=f32)
        deg2 = csum2[0, :k2] + 2.0
        dinv2 = lax.rsqrt(deg2)

        x3 = _dense_gcn(A2p, _feature_mm(xp2, Wd2), dinv2, bd2, True)

        score3 = jnp.tanh((x3 @ pw3) / jnp.linalg.norm(pw3))
        vals3, perm3, invp3 = _topk(score3, k3)
        xp3 = x3[perm3] * vals3[:, None]
        U3, W3 = _pool_adj(A2p, A2pT, perm3, k3p)
        A3p, csum3 = _matmul(
            U3, W3, dims=(1, 1), bm=_pick(k3p, 640), bn=_pick(k3p, 640),
            bk=_pick(k2p, 1280), acc_dtype=f32, out_dtype=f32,
            zero_diag=True, want_csum=True, want_ct=False)
        deg3 = csum3[0, :k3] + 2.0
        dinv3 = lax.rsqrt(deg3)

        x4 = _dense_gcn(A3p, _feature_mm(xp3, Wd3), dinv3, bd3, True)

        up = jnp.where((invp3 >= 0)[:, None], x4[jnp.maximum(invp3, 0)], 0.0)
        xu = _dense_gcn(A2p, _feature_mm(x3 + up, Wu1), dinv2, bu1, True)

        up = jnp.where((invp2 >= 0)[:, None], xu[jnp.maximum(invp2, 0)], 0.0)
        xu = _dense_gcn(A1p, _feature_mm(x2 + up, Wu2), dinv1, bu2, True)

        up = jnp.where((invp1 >= 0)[:, None], xu[jnp.maximum(invp1, 0)], 0.0)
        yf = _feature_mm(x1 + up, Wu3)
        zf = dinv0[:, None] * yf
        Sf = jnp.zeros((n0, yf.shape[1]), f32).at[d].add(zf[s])
        out = _gcn_epilogue(Sf, yf, dinv0, coef0, bu3, False)
        outs.append(out)

    return jnp.stack(outs, 0)
